# baseline (speedup 1.0000x reference)
import functools

import jax
import jax.numpy as jnp
from jax import lax
from jax.experimental import pallas as pl
from jax.experimental.pallas import tpu as pltpu, tpu_sc as plsc

N_NODES = 10000
N_EDGES = 320000
D = 128

NC = 2
NS = 16
NW = NC * NS
K = 128
CHUNKS = 80
EDGES_PER_W = K * CHUNKS
E_PAD = NW * EDGES_PER_W
GSIZE = 8
ROWS_PER_TILE = 640
N_PAD = NS * ROWS_PER_TILE
LAST_ROWS = N_NODES - 15 * ROWS_PER_TILE


def _sc_body(src_hbm, dst_hbm, feature_hbm, zeros_hbm, out_hbm,
             src_v, dst_v, buf_a, buf_b, acc,
             gsem_a, gsem_b, ssem_a, ssem_b):
    c = lax.axis_index("c")
    s = lax.axis_index("s")
    wid = s * NC + c

    pltpu.sync_copy(zeros_hbm, acc.at[pl.ds(s * ROWS_PER_TILE, ROWS_PER_TILE)])
    pltpu.sync_copy(src_hbm.at[wid], src_v)
    plsc.subcore_barrier()

    bufs = (buf_a, buf_b)
    gsems = (gsem_a, gsem_b)
    ssems = (ssem_a, ssem_b)
    pltpu.async_copy(feature_hbm.at[src_v.at[0]], buf_a, gsem_a)

    def chunk(j, carry):
        for p in range(2):
            @pl.when((((j + 1) & 1) == p) & (j >= 1))
            def _(p=p):
                pltpu.make_async_copy(
                    bufs[p], acc.at[dst_v.at[0]], ssems[p]).wait()

        @pl.when((j & (GSIZE - 1)) == 0)
        def _():
            g = j >> 3
            pltpu.sync_copy(dst_hbm.at[wid, pl.ds(g * GSIZE, GSIZE)], dst_v)

        for p in range(2):
            q = 1 - p

            @pl.when((j & 1) == p)
            def _(p=p, q=q):
                pltpu.make_async_copy(
                    feature_hbm.at[src_v.at[j]], bufs[p], gsems[p]).wait()
                pltpu.async_copy(
                    bufs[p], acc.at[dst_v.at[j & (GSIZE - 1)]], ssems[p],
                    add=True)

                @pl.when(j + 1 < CHUNKS)
                def _():
                    pltpu.async_copy(
                        feature_hbm.at[src_v.at[j + 1]], bufs[q], gsems[q])

        return carry

    lax.fori_loop(0, CHUNKS, chunk, 0)
    last = (CHUNKS - 1) % 2
    pltpu.make_async_copy(bufs[last], acc.at[dst_v.at[0]], ssems[last]).wait()
    plsc.subcore_barrier()

    base = c * N_NODES + s * ROWS_PER_TILE

    @pl.when(s < NS - 1)
    def _():
        pltpu.sync_copy(acc.at[pl.ds(s * ROWS_PER_TILE, ROWS_PER_TILE)],
                        out_hbm.at[pl.ds(base, ROWS_PER_TILE)])

    @pl.when(s == NS - 1)
    def _():
        pltpu.sync_copy(acc.at[pl.ds(s * ROWS_PER_TILE, LAST_ROWS)],
                        out_hbm.at[pl.ds(base, LAST_ROWS)])


_sc_aggregate = functools.partial(
    pl.kernel,
    out_type=jax.ShapeDtypeStruct((NC * N_NODES, D), jnp.float32),
    mesh=plsc.VectorSubcoreMesh(core_axis_name="c", subcore_axis_name="s"),
    scratch_types=[
        pltpu.VMEM((CHUNKS, K), jnp.int32),
        pltpu.VMEM((GSIZE, K), jnp.int32),
        pltpu.VMEM((K, D), jnp.float32),
        pltpu.VMEM((K, D), jnp.float32),
        pltpu.VMEM_SHARED((N_PAD, D), jnp.float32),
        pltpu.SemaphoreType.DMA,
        pltpu.SemaphoreType.DMA,
        pltpu.SemaphoreType.DMA,
        pltpu.SemaphoreType.DMA,
    ],
)(_sc_body)


def _mm_body(h_ref, wt_ref, b_ref, o_ref):
    h = h_ref[0] + h_ref[1]
    o_ref[...] = (
        jnp.dot(h, wt_ref[...], preferred_element_type=jnp.float32)
        + b_ref[...]
    )


def _tc_linear(h2, wt, b2):
    bm = 2000
    return pl.pallas_call(
        _mm_body,
        grid=(N_NODES // bm,),
        in_specs=[
            pl.BlockSpec((2, bm, D), lambda i: (0, i, 0)),
            pl.BlockSpec((D, D), lambda i: (0, 0)),
            pl.BlockSpec((1, D), lambda i: (0, 0)),
        ],
        out_specs=pl.BlockSpec((bm, D), lambda i: (i, 0)),
        out_shape=jax.ShapeDtypeStruct((N_NODES, D), jnp.float32),
    )(h2, wt, b2)


def kernel(edge_index, feature, W, b):
    edge_index = edge_index.astype(jnp.int32)
    n_fill = E_PAD - N_EDGES
    src3 = jnp.concatenate(
        [edge_index[0], jnp.zeros((n_fill,), jnp.int32)]).reshape(NW, CHUNKS, K)
    dst3 = jnp.concatenate(
        [edge_index[1], jnp.full((n_fill,), N_NODES, jnp.int32)]
    ).reshape(NW, CHUNKS, K)
    zeros = jnp.zeros((ROWS_PER_TILE, D), jnp.float32)
    hpart = _sc_aggregate(src3, dst3, feature, zeros)
    h2 = hpart.reshape(NC, N_NODES, D)
    return _tc_linear(h2, W.T, b.reshape(1, D))


# speedup vs baseline: 3.5835x; 3.5835x over previous
import functools

import jax
import jax.numpy as jnp
from jax import lax
from jax.experimental import pallas as pl
from jax.experimental.pallas import tpu as pltpu, tpu_sc as plsc

N_NODES = 10000
N_EDGES = 320000
D = 128

NC = 2
NS = 16
NW = NC * NS
K = 80
CHUNKS = 125
EDGES_PER_W = K * CHUNKS
GSIZE = 8
CHUNKS_PAD = 128
ROWS_PER_TILE = 640
N_PAD = NS * ROWS_PER_TILE
LAST_ROWS = N_NODES - 15 * ROWS_PER_TILE


def _sc_body(src_hbm, dst_hbm, feature_hbm, zeros_hbm, out_hbm,
             src_v, dst_v, buf_a, buf_b, buf_c, acc,
             gsem_a, gsem_b, gsem_c, ssem_a, ssem_b, ssem_c):
    c = lax.axis_index("c")
    s = lax.axis_index("s")
    wid = s * NC + c

    band = acc.at[pl.ds(s * ROWS_PER_TILE, ROWS_PER_TILE)]
    pltpu.async_copy(zeros_hbm, band, ssem_a)
    pltpu.async_copy(src_hbm.at[wid], src_v, ssem_b)
    pltpu.make_async_copy(src_hbm.at[wid], src_v, ssem_b).wait()

    bufs = (buf_a, buf_b, buf_c)
    gsems = (gsem_a, gsem_b, gsem_c)
    ssems = (ssem_a, ssem_b, ssem_c)
    pltpu.async_copy(feature_hbm.at[src_v.at[0]], buf_a, gsem_a)
    pltpu.async_copy(feature_hbm.at[src_v.at[1]], buf_b, gsem_b)
    pltpu.make_async_copy(zeros_hbm, band, ssem_a).wait()
    plsc.subcore_barrier()

    def chunk(j, carry):
        mprev = lax.rem(j + 2, 3)
        for p in range(3):
            @pl.when((mprev == p) & (j >= 1))
            def _(p=p):
                pltpu.make_async_copy(
                    bufs[p], acc.at[dst_v.at[0]], ssems[p]).wait()

        @pl.when((j & (GSIZE - 1)) == 0)
        def _():
            g = j >> 3
            pltpu.sync_copy(dst_hbm.at[wid, pl.ds(g * GSIZE, GSIZE)], dst_v)

        m = lax.rem(j, 3)
        for p in range(3):
            r = (p + 2) % 3

            @pl.when(m == p)
            def _(p=p, r=r):
                pltpu.make_async_copy(
                    feature_hbm.at[src_v.at[j]], bufs[p], gsems[p]).wait()
                pltpu.async_copy(
                    bufs[p], acc.at[dst_v.at[j & (GSIZE - 1)]], ssems[p],
                    add=True)

                @pl.when(j + 2 < CHUNKS)
                def _():
                    pltpu.async_copy(
                        feature_hbm.at[src_v.at[j + 2]], bufs[r], gsems[r])

        return carry

    lax.fori_loop(0, CHUNKS, chunk, 0)
    last = (CHUNKS - 1) % 3
    pltpu.make_async_copy(bufs[last], acc.at[dst_v.at[0]], ssems[last]).wait()
    plsc.subcore_barrier()

    base = c * N_NODES + s * ROWS_PER_TILE

    @pl.when(s < NS - 1)
    def _():
        pltpu.sync_copy(acc.at[pl.ds(s * ROWS_PER_TILE, ROWS_PER_TILE)],
                        out_hbm.at[pl.ds(base, ROWS_PER_TILE)])

    @pl.when(s == NS - 1)
    def _():
        pltpu.sync_copy(acc.at[pl.ds(s * ROWS_PER_TILE, LAST_ROWS)],
                        out_hbm.at[pl.ds(base, LAST_ROWS)])


_sc_aggregate = functools.partial(
    pl.kernel,
    out_type=jax.ShapeDtypeStruct((NC * N_NODES, D), jnp.float32),
    mesh=plsc.VectorSubcoreMesh(core_axis_name="c", subcore_axis_name="s"),
    scratch_types=[
        pltpu.VMEM((CHUNKS, K), jnp.int32),
        pltpu.VMEM((GSIZE, K), jnp.int32),
        pltpu.VMEM((K, D), jnp.float32),
        pltpu.VMEM((K, D), jnp.float32),
        pltpu.VMEM((K, D), jnp.float32),
        pltpu.VMEM_SHARED((N_PAD, D), jnp.float32),
        pltpu.SemaphoreType.DMA,
        pltpu.SemaphoreType.DMA,
        pltpu.SemaphoreType.DMA,
        pltpu.SemaphoreType.DMA,
        pltpu.SemaphoreType.DMA,
        pltpu.SemaphoreType.DMA,
    ],
)(_sc_body)


def _mm_body(h_ref, wt_ref, b_ref, o_ref):
    h = h_ref[0] + h_ref[1]
    o_ref[...] = (
        jnp.dot(h, wt_ref[...], preferred_element_type=jnp.float32)
        + b_ref[...]
    )


def _tc_linear(h2, wt, b2):
    bm = 2000
    return pl.pallas_call(
        _mm_body,
        grid=(N_NODES // bm,),
        in_specs=[
            pl.BlockSpec((2, bm, D), lambda i: (0, i, 0)),
            pl.BlockSpec((D, D), lambda i: (0, 0)),
            pl.BlockSpec((1, D), lambda i: (0, 0)),
        ],
        out_specs=pl.BlockSpec((bm, D), lambda i: (i, 0)),
        out_shape=jax.ShapeDtypeStruct((N_NODES, D), jnp.float32),
    )(h2, wt, b2)


def kernel(edge_index, feature, W, b):
    edge_index = edge_index.astype(jnp.int32)
    src3 = edge_index[0].reshape(NW, CHUNKS, K)
    dst3 = jnp.pad(edge_index[1].reshape(NW, CHUNKS, K),
                   ((0, 0), (0, CHUNKS_PAD - CHUNKS), (0, 0)))
    zeros = jnp.zeros((ROWS_PER_TILE, D), jnp.float32)
    hpart = _sc_aggregate(src3, dst3, feature, zeros)
    h2 = hpart.reshape(NC, N_NODES, D)
    return _tc_linear(h2, W.T, b.reshape(1, D))


# speedup vs baseline: 3.5964x; 1.0036x over previous
import functools

import jax
import jax.numpy as jnp
from jax import lax
from jax.experimental import pallas as pl
from jax.experimental.pallas import tpu as pltpu, tpu_sc as plsc

N_NODES = 10000
N_EDGES = 320000
D = 128

NC = 2
NS = 16
NW = NC * NS
K = 80
CHUNKS = 125
EDGES_PER_W = K * CHUNKS
GSIZE = 8
LAST_GROUP = (CHUNKS // GSIZE) * GSIZE
ROWS_PER_TILE = 640
N_PAD = NS * ROWS_PER_TILE
LAST_ROWS = N_NODES - 15 * ROWS_PER_TILE


def _sc_body(src_hbm, dst_hbm, feature_hbm, zeros_hbm, out_hbm,
             src_v, dst_v, buf_a, buf_b, buf_c, acc,
             gsem_a, gsem_b, gsem_c, ssem_a, ssem_b, ssem_c):
    c = lax.axis_index("c")
    s = lax.axis_index("s")
    wid = s * NC + c

    band = acc.at[pl.ds(s * ROWS_PER_TILE, ROWS_PER_TILE)]
    pltpu.async_copy(zeros_hbm, band, ssem_a)
    pltpu.async_copy(src_hbm.at[wid], src_v, ssem_b)
    pltpu.make_async_copy(src_hbm.at[wid], src_v, ssem_b).wait()

    bufs = (buf_a, buf_b, buf_c)
    gsems = (gsem_a, gsem_b, gsem_c)
    ssems = (ssem_a, ssem_b, ssem_c)
    pltpu.async_copy(feature_hbm.at[src_v.at[0]], buf_a, gsem_a)
    pltpu.async_copy(feature_hbm.at[src_v.at[1]], buf_b, gsem_b)
    pltpu.make_async_copy(zeros_hbm, band, ssem_a).wait()
    plsc.subcore_barrier()

    def chunk(j, carry):
        mprev = lax.rem(j + 2, 3)
        for p in range(3):
            @pl.when((mprev == p) & (j >= 1))
            def _(p=p):
                pltpu.make_async_copy(
                    bufs[p], acc.at[dst_v.at[0]], ssems[p]).wait()

        @pl.when(((j & (GSIZE - 1)) == 0) & (j < LAST_GROUP))
        def _():
            pltpu.sync_copy(
                dst_hbm.at[wid, pl.ds((j >> 3) * GSIZE, GSIZE)], dst_v)

        @pl.when(j == LAST_GROUP)
        def _():
            pltpu.sync_copy(
                dst_hbm.at[wid, pl.ds(LAST_GROUP, CHUNKS - LAST_GROUP)],
                dst_v.at[pl.ds(0, CHUNKS - LAST_GROUP)])

        m = lax.rem(j, 3)
        for p in range(3):
            r = (p + 2) % 3

            @pl.when(m == p)
            def _(p=p, r=r):
                pltpu.make_async_copy(
                    feature_hbm.at[src_v.at[j]], bufs[p], gsems[p]).wait()
                pltpu.async_copy(
                    bufs[p], acc.at[dst_v.at[j & (GSIZE - 1)]], ssems[p],
                    add=True)

                @pl.when(j + 2 < CHUNKS)
                def _():
                    pltpu.async_copy(
                        feature_hbm.at[src_v.at[j + 2]], bufs[r], gsems[r])

        return carry

    lax.fori_loop(0, CHUNKS, chunk, 0)
    last = (CHUNKS - 1) % 3
    pltpu.make_async_copy(bufs[last], acc.at[dst_v.at[0]], ssems[last]).wait()
    plsc.subcore_barrier()

    base = c * N_NODES + s * ROWS_PER_TILE

    @pl.when(s < NS - 1)
    def _():
        pltpu.sync_copy(acc.at[pl.ds(s * ROWS_PER_TILE, ROWS_PER_TILE)],
                        out_hbm.at[pl.ds(base, ROWS_PER_TILE)])

    @pl.when(s == NS - 1)
    def _():
        pltpu.sync_copy(acc.at[pl.ds(s * ROWS_PER_TILE, LAST_ROWS)],
                        out_hbm.at[pl.ds(base, LAST_ROWS)])


_sc_aggregate = functools.partial(
    pl.kernel,
    out_type=jax.ShapeDtypeStruct((NC * N_NODES, D), jnp.float32),
    mesh=plsc.VectorSubcoreMesh(core_axis_name="c", subcore_axis_name="s"),
    scratch_types=[
        pltpu.VMEM((CHUNKS, K), jnp.int32),
        pltpu.VMEM((GSIZE, K), jnp.int32),
        pltpu.VMEM((K, D), jnp.float32),
        pltpu.VMEM((K, D), jnp.float32),
        pltpu.VMEM((K, D), jnp.float32),
        pltpu.VMEM_SHARED((N_PAD, D), jnp.float32),
        pltpu.SemaphoreType.DMA,
        pltpu.SemaphoreType.DMA,
        pltpu.SemaphoreType.DMA,
        pltpu.SemaphoreType.DMA,
        pltpu.SemaphoreType.DMA,
        pltpu.SemaphoreType.DMA,
    ],
)(_sc_body)


def _mm_body(h_ref, wt_ref, b_ref, o_ref):
    h = h_ref[0] + h_ref[1]
    o_ref[...] = (
        jnp.dot(h, wt_ref[...], preferred_element_type=jnp.float32)
        + b_ref[...]
    )


def _tc_linear(h2, wt, b2):
    bm = 2000
    return pl.pallas_call(
        _mm_body,
        grid=(N_NODES // bm,),
        in_specs=[
            pl.BlockSpec((2, bm, D), lambda i: (0, i, 0)),
            pl.BlockSpec((D, D), lambda i: (0, 0)),
            pl.BlockSpec((1, D), lambda i: (0, 0)),
        ],
        out_specs=pl.BlockSpec((bm, D), lambda i: (i, 0)),
        out_shape=jax.ShapeDtypeStruct((N_NODES, D), jnp.float32),
    )(h2, wt, b2)


def kernel(edge_index, feature, W, b):
    edge_index = edge_index.astype(jnp.int32)
    src3 = edge_index[0].reshape(NW, CHUNKS, K)
    dst3 = edge_index[1].reshape(NW, CHUNKS, K)
    zeros = jnp.zeros((ROWS_PER_TILE, D), jnp.float32)
    hpart = _sc_aggregate(src3, dst3, feature, zeros)
    h2 = hpart.reshape(NC, N_NODES, D)
    return _tc_linear(h2, W.T, b.reshape(1, D))
